# Initial kernel scaffold; baseline (speedup 1.0000x reference)
import jax
import jax.numpy as jnp
from jax.experimental import pallas as pl


def kernel(q, k, v, w1, b1, w2, b2):
    raise NotImplementedError("write your pallas kernel here")


# speedup vs baseline: 125.9142x; 125.9142x over previous
import jax
import jax.numpy as jnp
from jax.experimental import pallas as pl
from jax.experimental.pallas import tpu as pltpu

_N = 4096
_CH = 128
_NC = _N // _CH
_PHI_TILE = 512


def _body(q_ref, k_ref, v_ref, w1_ref, b1_ref, w2_ref, b2_ref, o_ref,
          qp_ref, kp_ref, kv_ref):
    F = w1_ref.shape[1]
    D = v_ref.shape[-1]
    w1 = w1_ref[...]
    w2 = w2_ref[...]
    b1 = b1_ref[...]
    b2 = b2_ref[...]

    for t in range(_N // _PHI_TILE):
        sl = slice(t * _PHI_TILE, (t + 1) * _PHI_TILE)
        for src, dst in ((q_ref, qp_ref), (k_ref, kp_ref)):
            x = src[0, sl, :]
            h = jnp.dot(x, w1, preferred_element_type=jnp.float32) + b1
            h = h * jax.nn.sigmoid(h)
            p = jnp.dot(h, w2, preferred_element_type=jnp.float32) + b2
            dst[sl, :] = p

    tot = jnp.zeros((F, D), jnp.float32)
    for c in range(_NC):
        sl = slice(c * _CH, (c + 1) * _CH)
        kc = kp_ref[sl, :]
        vc = v_ref[0, sl, :]
        kv = jax.lax.dot_general(kc, vc, (((0,), (0,)), ((), ())),
                                 preferred_element_type=jnp.float32)
        kv_ref[c] = kv
        tot = tot + kv

    ii = jax.lax.broadcasted_iota(jnp.int32, (_CH, _CH), 0)
    jj = jax.lax.broadcasted_iota(jnp.int32, (_CH, _CH), 1)
    low = ii >= jj
    up = jj >= ii
    rowpos = jax.lax.broadcasted_iota(
        jnp.int32, (_CH, D), 0).astype(jnp.float32)
    wf = jnp.zeros((F, D), jnp.float32)
    for c in range(_NC):
        sl = slice(c * _CH, (c + 1) * _CH)
        qc = qp_ref[sl, :]
        kc = kp_ref[sl, :]
        vc = v_ref[0, sl, :]
        kv = kv_ref[c]
        wr = tot - wf - kv
        s = jax.lax.dot_general(qc, kc, (((1,), (1,)), ((), ())),
                                preferred_element_type=jnp.float32)
        s_low = jnp.where(low, s, 0.0)
        s_up = jnp.where(up, s, 0.0)
        fwd = (jnp.dot(s_low, vc, preferred_element_type=jnp.float32)
               + jnp.dot(qc, wf, preferred_element_type=jnp.float32))
        rev = (jnp.dot(s_up, vc, preferred_element_type=jnp.float32)
               + jnp.dot(qc, wr, preferred_element_type=jnp.float32))
        nn = rowpos + float(c * _CH)
        o_ref[0, sl, :] = fwd * (1.0 / (nn + 1.0)) + rev * (1.0 / (float(_N) - nn))
        wf = wf + kv


def kernel(q, k, v, w1, b1, w2, b2):
    B, H, n, D = q.shape
    F = w1.shape[1]
    BH = B * H
    qf = q.reshape(BH, n, D)
    kf = k.reshape(BH, n, D)
    vf = v.reshape(BH, n, D)
    out = pl.pallas_call(
        _body,
        out_shape=jax.ShapeDtypeStruct((BH, n, D), jnp.float32),
        grid=(BH,),
        in_specs=[
            pl.BlockSpec((1, n, D), lambda b: (b, 0, 0)),
            pl.BlockSpec((1, n, D), lambda b: (b, 0, 0)),
            pl.BlockSpec((1, n, D), lambda b: (b, 0, 0)),
            pl.BlockSpec((D, F), lambda b: (0, 0)),
            pl.BlockSpec((1, F), lambda b: (0, 0)),
            pl.BlockSpec((F, F), lambda b: (0, 0)),
            pl.BlockSpec((1, F), lambda b: (0, 0)),
        ],
        out_specs=pl.BlockSpec((1, n, D), lambda b: (b, 0, 0)),
        scratch_shapes=[
            pltpu.VMEM((n, F), jnp.float32),
            pltpu.VMEM((n, F), jnp.float32),
            pltpu.VMEM((_NC, F, D), jnp.float32),
        ],
        compiler_params=pltpu.CompilerParams(
            dimension_semantics=("parallel",),
            vmem_limit_bytes=50 * 1024 * 1024,
        ),
        name="ttrflux_fused",
    )(qf, kf, vf, w1, b1.reshape(1, F), w2, b2.reshape(1, F))
    return out.reshape(B, H, n, D)


# speedup vs baseline: 129.7951x; 1.0308x over previous
import jax
import jax.numpy as jnp
from jax.experimental import pallas as pl
from jax.experimental.pallas import tpu as pltpu

_N = 4096
_CH = 128
_NC = _N // _CH
_PHI_TILE = 512


def _body(q_ref, k_ref, v_ref, w1_ref, b1_ref, w2_ref, b2_ref, o_ref,
          qp_ref, kp_ref, kv_ref):
    F = w1_ref.shape[1]
    D = v_ref.shape[-1]
    w1 = w1_ref[...]
    w2 = w2_ref[...]
    b1 = b1_ref[...]
    b2 = b2_ref[...]

    for t in range(_N // _PHI_TILE):
        sl = slice(t * _PHI_TILE, (t + 1) * _PHI_TILE)
        for src, dst in ((q_ref, qp_ref), (k_ref, kp_ref)):
            x = src[0, sl, :]
            h = jnp.dot(x, w1, preferred_element_type=jnp.float32) + b1
            h = h * (1.0 / (1.0 + jnp.exp(-h)))
            p = jnp.dot(h, w2, preferred_element_type=jnp.float32) + b2
            dst[sl, :] = p

    tot = jnp.zeros((D, F), jnp.float32)
    for c in range(_NC):
        sl = slice(c * _CH, (c + 1) * _CH)
        kc = kp_ref[sl, :]
        vc = v_ref[0, sl, :]
        kv = jax.lax.dot_general(vc, kc, (((0,), (0,)), ((), ())),
                                 preferred_element_type=jnp.float32)
        kv_ref[c] = kv
        tot = tot + kv

    ii = jax.lax.broadcasted_iota(jnp.int32, (_CH, _CH), 0)
    jj = jax.lax.broadcasted_iota(jnp.int32, (_CH, _CH), 1)
    low = ii >= jj
    up = jj >= ii
    rowf_s = jax.lax.broadcasted_iota(
        jnp.int32, (_CH, _CH), 0).astype(jnp.float32)
    rowf_q = jax.lax.broadcasted_iota(
        jnp.int32, (_CH, F), 0).astype(jnp.float32)
    wf = jnp.zeros((D, F), jnp.float32)
    for c in range(_NC):
        sl = slice(c * _CH, (c + 1) * _CH)
        qc = qp_ref[sl, :]
        kc = kp_ref[sl, :]
        vc = v_ref[0, sl, :]
        kv = kv_ref[c]
        wr = tot - wf - kv
        s = jax.lax.dot_general(qc, kc, (((1,), (1,)), ((), ())),
                                preferred_element_type=jnp.float32)
        base = float(c * _CH)
        sf_s = 1.0 / (rowf_s + (base + 1.0))
        sr_s = 1.0 / ((float(_N) - base) - rowf_s)
        wmask = jnp.where(low, sf_s, 0.0) + jnp.where(up, sr_s, 0.0)
        intra = jnp.dot(s * wmask, vc, preferred_element_type=jnp.float32)
        finter = jax.lax.dot_general(qc, wf, (((1,), (1,)), ((), ())),
                                     preferred_element_type=jnp.float32)
        rinter = jax.lax.dot_general(qc, wr, (((1,), (1,)), ((), ())),
                                     preferred_element_type=jnp.float32)
        nn = rowf_q[:, :D] + base
        o_ref[0, sl, :] = (intra + finter * (1.0 / (nn + 1.0))
                           + rinter * (1.0 / (float(_N) - nn)))
        wf = wf + kv


def kernel(q, k, v, w1, b1, w2, b2):
    B, H, n, D = q.shape
    F = w1.shape[1]
    BH = B * H
    qf = q.reshape(BH, n, D)
    kf = k.reshape(BH, n, D)
    vf = v.reshape(BH, n, D)
    out = pl.pallas_call(
        _body,
        out_shape=jax.ShapeDtypeStruct((BH, n, D), jnp.float32),
        grid=(BH,),
        in_specs=[
            pl.BlockSpec((1, n, D), lambda b: (b, 0, 0)),
            pl.BlockSpec((1, n, D), lambda b: (b, 0, 0)),
            pl.BlockSpec((1, n, D), lambda b: (b, 0, 0)),
            pl.BlockSpec((D, F), lambda b: (0, 0)),
            pl.BlockSpec((1, F), lambda b: (0, 0)),
            pl.BlockSpec((F, F), lambda b: (0, 0)),
            pl.BlockSpec((1, F), lambda b: (0, 0)),
        ],
        out_specs=pl.BlockSpec((1, n, D), lambda b: (b, 0, 0)),
        scratch_shapes=[
            pltpu.VMEM((n, F), jnp.float32),
            pltpu.VMEM((n, F), jnp.float32),
            pltpu.VMEM((_NC, D, F), jnp.float32),
        ],
        compiler_params=pltpu.CompilerParams(
            dimension_semantics=("parallel",),
            vmem_limit_bytes=50 * 1024 * 1024,
        ),
        name="ttrflux_fused",
    )(qf, kf, vf, w1, b1.reshape(1, F), w2, b2.reshape(1, F))
    return out.reshape(B, H, n, D)
